# baseline (speedup 1.0000x reference)
import functools

import jax
import jax.numpy as jnp
from jax import lax
from jax.experimental import pallas as pl
from jax.experimental.pallas import tpu as pltpu
from jax.experimental.pallas import tpu_sc as plsc

NJ = 10
LANE = 128


@functools.lru_cache(maxsize=None)
def _make_kernel(b, s, dm):
    info = plsc.get_sparse_core_info()
    nc, ns = info.num_cores, info.num_subcores
    nw = nc * ns
    n_blocks = b // LANE
    cb_per_w = n_blocks // nw
    n_groups = s // NJ
    gr = dm // 8
    mesh = plsc.VectorSubcoreMesh(core_axis_name="c", subcore_axis_name="s")

    @functools.partial(
        pl.kernel,
        mesh=mesh,
        compiler_params=pltpu.CompilerParams(
            use_tc_tiling_on_sc=False, needs_layout_passes=False
        ),
        out_type=jax.ShapeDtypeStruct((s, gr, n_blocks, 8 * LANE), jnp.float32),
        scratch_types=[
            pltpu.VMEM((2, NJ, LANE), jnp.int32),
            pltpu.VMEM((2, NJ, LANE, dm), jnp.float32),
            pltpu.VMEM((gr, NJ, 8 * LANE), jnp.float32),
            pltpu.SemaphoreType.DMA,
            pltpu.SemaphoreType.DMA,
            pltpu.SemaphoreType.DMA,
        ],
    )
    def sc_gather(xt_hbm, table_hbm, out_hbm, idx_v, rows_v, t_v, sem_a, sem_b, sem_w):
        wid = lax.axis_index("s") * nc + lax.axis_index("c")
        c0 = wid * cb_per_w
        iota = lax.iota(jnp.int32, 16)
        sems = (sem_a, sem_b)

        def stage_and_fire(cg, q, p):
            j0 = q * NJ
            pltpu.sync_copy(
                xt_hbm.at[pl.ds(j0, NJ), pl.ds(cg * LANE, LANE)], idx_v.at[p]
            )
            return [
                pltpu.async_copy(
                    table_hbm.at[idx_v.at[p, u]], rows_v.at[p, u], sems[p]
                )
                for u in range(NJ)
            ]

        def cblock(cb, carry):
            cg = c0 + cb
            pending = stage_and_fire(cg, 0, 0)
            pending_w = []
            for q in range(n_groups):
                p = q & 1
                nxt = (
                    stage_and_fire(cg, q + 1, 1 - p)
                    if q + 1 < n_groups
                    else []
                )
                for cp in pending:
                    cp.wait()
                pending = nxt
                for wh in pending_w:
                    wh.wait()

                def transpose_unit(u, c_):
                    for m in range(8):
                        rowv = iota + m * 16
                        for g in range(gr):
                            for ss in range(8):
                                vals = plsc.load_gather(
                                    rows_v.at[p, u],
                                    [rowv, jnp.full((16,), g * 8 + ss, jnp.int32)],
                                )
                                t_v[g, u, pl.ds(ss * LANE + m * 16, 16)] = vals
                    return c_

                lax.fori_loop(0, NJ, transpose_unit, 0)
                pending_w = [
                    pltpu.async_copy(
                        t_v.at[g],
                        out_hbm.at[pl.ds(q * NJ, NJ), g, cg],
                        sem_w,
                    )
                    for g in range(gr)
                ]
            for wh in pending_w:
                wh.wait()
            return carry

        lax.fori_loop(0, cb_per_w, cblock, 0)

    return sc_gather


def kernel(x, W):
    b, s = x.shape
    dm = W.shape[1]
    xt = x.astype(jnp.int32).T
    out = _make_kernel(b, s, dm)(xt, W)
    return (
        out.reshape(s, dm // 8, b // LANE, 8, LANE)
        .transpose(2, 4, 0, 1, 3)
        .reshape(b, s, dm)
    )


# speedup vs baseline: 1.4214x; 1.4214x over previous
import functools

import jax
import jax.numpy as jnp
from jax import lax
from jax.experimental import pallas as pl
from jax.experimental.pallas import tpu as pltpu
from jax.experimental.pallas import tpu_sc as plsc

NJ = 10
LANE = 128


@functools.lru_cache(maxsize=None)
def _make_kernel(b, s, dm):
    info = plsc.get_sparse_core_info()
    nc, ns = info.num_cores, info.num_subcores
    nw = nc * ns
    n_blocks = b // LANE
    cb_per_w = n_blocks // nw
    n_groups = s // NJ
    gr = dm // 8
    mesh = plsc.VectorSubcoreMesh(core_axis_name="c", subcore_axis_name="s")

    @functools.partial(
        pl.kernel,
        mesh=mesh,
        compiler_params=pltpu.CompilerParams(
            use_tc_tiling_on_sc=False, needs_layout_passes=False
        ),
        out_type=jax.ShapeDtypeStruct((s, gr, n_blocks, 8 * LANE), jnp.float32),
        scratch_types=[
            pltpu.VMEM((2, NJ, LANE), jnp.int32),
            pltpu.VMEM((2, NJ, LANE, dm), jnp.float32),
            pltpu.VMEM((gr, NJ, 8 * LANE), jnp.float32),
            pltpu.SemaphoreType.DMA,
            pltpu.SemaphoreType.DMA,
            pltpu.SemaphoreType.DMA,
        ],
    )
    def sc_gather(xt_hbm, table_hbm, out_hbm, idx_v, rows_v, t_v, sem_a, sem_b, sem_w):
        wid = lax.axis_index("s") * nc + lax.axis_index("c")
        c0 = wid * cb_per_w
        iota = lax.iota(jnp.int32, 16)
        sems = (sem_a, sem_b)

        def stage_and_fire(cg, q, p):
            j0 = q * NJ
            pltpu.sync_copy(
                xt_hbm.at[pl.ds(j0, NJ), pl.ds(cg * LANE, LANE)], idx_v.at[p]
            )
            return [
                pltpu.async_copy(
                    table_hbm.at[idx_v.at[p, u]], rows_v.at[p, u], sems[p]
                )
                for u in range(NJ)
            ]

        def cblock(cb, carry):
            cg = c0 + cb
            pending = stage_and_fire(cg, 0, 0)
            pending_w = []
            for q in range(n_groups):
                p = q & 1
                nxt = (
                    stage_and_fire(cg, q + 1, 1 - p)
                    if q + 1 < n_groups
                    else []
                )
                for cp in pending:
                    cp.wait()
                pending = nxt
                for wh in pending_w:
                    wh.wait()

                @plsc.parallel_loop(0, NJ * 8, 1, unroll=2)
                def transpose_iter(k):
                    u = k // 8
                    m = k - u * 8
                    rowv = iota + m * 16
                    for g in range(gr):
                        for ss in range(8):
                            vals = plsc.load_gather(
                                rows_v.at[p, u],
                                [rowv, jnp.full((16,), g * 8 + ss, jnp.int32)],
                            )
                            t_v[g, u, pl.ds(ss * LANE + m * 16, 16)] = vals
                pending_w = [
                    pltpu.async_copy(
                        t_v.at[g],
                        out_hbm.at[pl.ds(q * NJ, NJ), g, cg],
                        sem_w,
                    )
                    for g in range(gr)
                ]
            for wh in pending_w:
                wh.wait()
            return carry

        lax.fori_loop(0, cb_per_w, cblock, 0)

    return sc_gather


def kernel(x, W):
    b, s = x.shape
    dm = W.shape[1]
    xt = x.astype(jnp.int32).T
    out = _make_kernel(b, s, dm)(xt, W)
    return (
        out.reshape(s, dm // 8, b // LANE, 8, LANE)
        .transpose(2, 4, 0, 1, 3)
        .reshape(b, s, dm)
    )
